# speedup vs baseline: 2.2278x; 2.2278x over previous
import functools

import jax
import jax.numpy as jnp
from jax import lax
from jax.experimental import pallas as pl
from jax.experimental.pallas import tpu as pltpu
from jax.experimental.pallas import tpu_sc as plsc

_NC, _NS = 2, 16
_NW = _NC * _NS
_CHUNK = 128


def _sc_gather(user2d, rsid2d, users_table, rsids_table, B, L):
    b_per_w = B // _NW
    n_chunks = b_per_w // _CHUNK
    r_per_w = L // _NS

    mesh = plsc.VectorSubcoreMesh(core_axis_name="c", subcore_axis_name="s")

    @functools.partial(
        pl.kernel,
        out_type=(
            jax.ShapeDtypeStruct((B, L), jnp.float32),
            jax.ShapeDtypeStruct((L, L), jnp.float32),
        ),
        mesh=mesh,
        scratch_types=[
            pltpu.VMEM((n_chunks, _CHUNK), jnp.int32),
            pltpu.VMEM((b_per_w, L), jnp.float32),
            pltpu.VMEM((r_per_w,), jnp.int32),
            pltpu.VMEM((r_per_w, L), jnp.float32),
            pltpu.SemaphoreType.DMA,
        ],
    )
    def gather_kernel(user_hbm, rsid_hbm, utab_hbm, rtab_hbm, u_out, r_out,
                      uidx_v, urows_v, ridx_v, rrows_v, sem):
        wid = lax.axis_index("s") * _NC + lax.axis_index("c")
        base = wid * b_per_w

        pltpu.sync_copy(user_hbm.at[wid], uidx_v)
        copies = []
        for j in range(n_chunks):
            copies.append(pltpu.async_copy(
                utab_hbm.at[uidx_v.at[j]],
                urows_v.at[pl.ds(j * _CHUNK, _CHUNK)],
                sem,
            ))
        for c in copies:
            c.wait()
        pltpu.sync_copy(urows_v, u_out.at[pl.ds(base, b_per_w)])

        @pl.when(wid < _NS)
        def _():
            pltpu.sync_copy(rsid_hbm.at[wid], ridx_v)
            pltpu.async_copy(rtab_hbm.at[ridx_v], rrows_v, sem).wait()
            pltpu.sync_copy(rrows_v, r_out.at[pl.ds(wid * r_per_w, r_per_w)])

    return gather_kernel(user2d, rsid2d, users_table, rsids_table)


def _tc_matmul(u, r, B, L, tile_b=2048):
    def mm_body(u_ref, r_ref, o_ref):
        o_ref[...] = jnp.dot(u_ref[...], r_ref[...],
                             preferred_element_type=jnp.float32)

    return pl.pallas_call(
        mm_body,
        grid=(B // tile_b,),
        in_specs=[
            pl.BlockSpec((tile_b, L), lambda i: (i, 0)),
            pl.BlockSpec((L, L), lambda i: (0, 0)),
        ],
        out_specs=pl.BlockSpec((tile_b, L), lambda i: (i, 0)),
        out_shape=jax.ShapeDtypeStruct((B, L), jnp.float32),
    )(u, r)


def kernel(user, rsid, users_table, rsids_table):
    B = user.shape[0]
    L = rsids_table.shape[1]
    user2d = user.reshape(_NW, B // _NW // _CHUNK, _CHUNK)
    rsid2d = rsid.reshape(_NS, L // _NS)
    u, r = _sc_gather(user2d, rsid2d, users_table, rsids_table, B, L)
    return _tc_matmul(u, r, B, L)
